# speedup vs baseline: 1.3125x; 1.3125x over previous
import functools

import jax
import jax.numpy as jnp
from jax import lax
from jax.experimental import pallas as pl
from jax.experimental.pallas import tpu as pltpu
from jax.experimental.pallas import tpu_sc as plsc

_DIM = 1024
_N = 1024 * 50
_NC, _NS = 2, 16
_NW = _NC * _NS
_BPW = _N // _NW
_CHUNK = 40
_NCHUNKS = _BPW // _CHUNK
_NBUF = 2
_MAIN = _NCHUNKS - _NBUF

_mesh = plsc.VectorSubcoreMesh(
    core_axis_name="c", subcore_axis_name="s",
    num_cores=_NC, num_subcores=_NS,
)


@functools.partial(
    pl.kernel,
    out_type=jax.ShapeDtypeStruct((_N, _DIM), jnp.float32),
    mesh=_mesh,
    scratch_types=[
        pltpu.VMEM((_BPW,), jnp.int32),
        pltpu.VMEM((_NBUF, _CHUNK, _DIM), jnp.float32),
        pltpu.SemaphoreType.DMA((_NBUF,)),
        pltpu.SemaphoreType.DMA((_NBUF,)),
    ],
)
def _gather_kernel(idx_hbm, table_hbm, out_hbm, idx_v, rows_v, gsem, wsem):
    wid = lax.axis_index("s") * _NC + lax.axis_index("c")
    base = wid * _BPW
    pltpu.sync_copy(idx_hbm.at[pl.ds(base, _BPW)], idx_v)

    def start_gather(g, b):
        pltpu.async_copy(
            table_hbm.at[idx_v.at[pl.ds(g * _CHUNK, _CHUNK)]],
            rows_v.at[b], gsem.at[b])

    def wait_gather(b):
        pltpu.make_async_copy(
            table_hbm.at[idx_v.at[pl.ds(0, _CHUNK)]],
            rows_v.at[b], gsem.at[b]).wait()

    def start_write(g, b):
        pltpu.async_copy(
            rows_v.at[b],
            out_hbm.at[pl.ds(base + g * _CHUNK, _CHUNK)], wsem.at[b])

    def wait_write(b):
        pltpu.make_async_copy(
            rows_v.at[b],
            out_hbm.at[pl.ds(base, _CHUNK)], wsem.at[b]).wait()

    for b in range(_NBUF):
        start_gather(b, b)

    @pl.loop(0, _MAIN, step=_NBUF)
    def _steady(i):
        for b in range(_NBUF):
            g = i + b
            wait_gather(b)
            start_write(g, b)
            wait_write(b)
            start_gather(g + _NBUF, b)

    for b in range(_NBUF):
        g = _MAIN + b
        wait_gather(b)
        start_write(g, b)
        wait_write(b)


def kernel(input, table):
    idx = input.reshape(-1).astype(jnp.int32)
    out = _gather_kernel(idx, table)
    return out.reshape(input.shape + (table.shape[1],))
